# Initial kernel scaffold; baseline (speedup 1.0000x reference)
import jax
import jax.numpy as jnp
from jax.experimental import pallas as pl


def kernel(x, edge_index_in, edge_index_out, params):
    raise NotImplementedError("write your pallas kernel here")


# speedup vs baseline: 1.1828x; 1.1828x over previous
import functools
import numpy as np
import jax
import jax.numpy as jnp
from jax import lax
from jax.experimental import pallas as pl
from jax.experimental.pallas import tpu as pltpu

N = 10000
F = 128
AVG_LOG = float(np.log(33.0))


def _pna_factored(x, edge_index, p):
    src = edge_index[0]
    dst = edge_index[1]
    n = x.shape[0]
    Wd = p['pre_W'][:F]
    Ws = p['pre_W'][F:]
    d = x @ Wd + p['pre_b']
    s = x @ Ws
    sv = s[src]
    cnt = jax.ops.segment_sum(jnp.ones((src.shape[0],), jnp.float32), dst, num_segments=n)
    S = jax.ops.segment_sum(sv, dst, num_segments=n)
    S2 = jax.ops.segment_sum(sv * sv, dst, num_segments=n)
    MN = jax.ops.segment_min(sv, dst, num_segments=n)
    MX = jax.ops.segment_max(sv, dst, num_segments=n)
    cnt_c = jnp.maximum(cnt, 1.0)[:, None]
    sum_h = cnt[:, None] * d + S
    mean = sum_h / cnt_c
    mean2 = (cnt[:, None] * d * d + 2.0 * d * S + S2) / cnt_c
    std = jnp.sqrt(jax.nn.relu(mean2 - mean * mean) + 1e-5)
    mnv = d + MN
    mxv = d + MX
    mnv = jnp.where(jnp.isfinite(mnv), mnv, 0.0)
    mxv = jnp.where(jnp.isfinite(mxv), mxv, 0.0)
    logdeg = jnp.log(jnp.maximum(cnt, 1.0) + 1.0)[:, None]
    amp = logdeg / AVG_LOG
    att = AVG_LOG / logdeg
    pw = p['post_W']
    Px, P1, P2, P3 = pw[:F], pw[F:F + 640], pw[F + 640:F + 1280], pw[F + 1280:]
    agg = jnp.concatenate([mean, sum_h, std, mnv, mxv], axis=-1)
    out = x @ Px + agg @ P1 + amp * (agg @ P2) + att * (agg @ P3) + p['post_b']
    return out @ p['lin_W'] + p['lin_b']


def _proj_kernel(x_ref, w_ref, b_ref, o_ref):
    o_ref[...] = x_ref[...] @ w_ref[...] + b_ref[...]


def _final_proj(x, W, b):
    Wp = jnp.zeros((F, 128), jnp.float32).at[:, 0].set(W[:, 0])
    bp = jnp.zeros((1, 128), jnp.float32).at[0, 0].set(b[0])
    bn = 400
    out = pl.pallas_call(
        _proj_kernel,
        grid=(N // bn,),
        in_specs=[
            pl.BlockSpec((bn, F), lambda i: (i, 0)),
            pl.BlockSpec((F, 128), lambda i: (0, 0)),
            pl.BlockSpec((1, 128), lambda i: (0, 0)),
        ],
        out_specs=pl.BlockSpec((bn, 128), lambda i: (i, 0)),
        out_shape=jax.ShapeDtypeStruct((N, 128), jnp.float32),
    )(x, Wp, bp)
    return out[:, 0]


def kernel(x, edge_index_in, edge_index_out, params):
    for layer in params['layers']:
        h_in = jax.nn.relu(_pna_factored(x, edge_index_in, layer['supply']))
        h_out = jax.nn.relu(_pna_factored(x, edge_index_out, layer['demand']))
        cw = layer['combine']['W']
        C0, C1, C2 = cw[:F], cw[F:2 * F], cw[2 * F:]
        x = jax.nn.relu(x @ C0 + h_in @ C1 + h_out @ C2 + layer['combine']['b'])
    return _final_proj(x, params['out']['W'], params['out']['b'])


# speedup vs baseline: 4.9484x; 4.1836x over previous
import functools
import numpy as np
import jax
import jax.numpy as jnp
from jax import lax
from jax.experimental import pallas as pl
from jax.experimental.pallas import tpu as pltpu
from jax.experimental.pallas import tpu_sc as plsc

N = 10000
E = 320000
F = 128
AVG_LOG = float(np.log(33.0))

NW = 32
NB = 64
NBW = 160
NPAD = NB * NBW
ECAP = E
CH = 2048
SCH = 4096
WIN = 64
INF = float(np.finfo(np.float32).max)

_mesh = plsc.VectorSubcoreMesh(core_axis_name="c", subcore_axis_name="s")
_sc_params = pltpu.CompilerParams(needs_layout_passes=False)


def _wid():
    return lax.axis_index("s") * 2 + lax.axis_index("c")


def _bin_body(ei, bsrc, bdst, woff, srcs_v, dsts_v,
              src_st0, dst_st0, src_st1, dst_st1, cnt_v):
    w = _wid()
    ones_f = jnp.ones((16,), jnp.float32)
    zeros_f = jnp.zeros((16,), jnp.float32)
    lane = plsc.cumsum(ones_f).astype(jnp.int32) - 1
    stages = ((src_st0, dst_st0), (src_st1, dst_st1))

    def scan_chunk(ci, carry):
        eb = pl.multiple_of(ci * SCH, SCH)
        pltpu.sync_copy(ei.at[0, pl.ds(eb, SCH)], srcs_v)
        pltpu.sync_copy(ei.at[1, pl.ds(eb, SCH)], dsts_v)

        def group(g, carry2):
            cur0, fl0, cur1, fl1 = carry2
            sv = srcs_v[pl.ds(g * 16, 16)]
            dv = dsts_v[pl.ds(g * 16, 16)]
            out = []
            for r, (cur, fl) in enumerate(((cur0, fl0), (cur1, fl1))):
                lo = w * 2 * NBW + r * NBW
                src_st, dst_st = stages[r]
                m = jnp.logical_and(dv >= lo, dv < lo + NBW)
                mf = jnp.where(m, ones_f, zeros_f)
                csum = plsc.cumsum(mf)
                pos = cur + (csum - mf).astype(jnp.int32)
                plsc.store_scatter(src_st, [pos], sv, mask=m)
                plsc.store_scatter(dst_st, [pos], dv - lo, mask=m)
                cur = cur + csum[15].astype(jnp.int32)

                @pl.when(cur >= CH)
                def _flush():
                    fo = pl.multiple_of((w * 2 + r) * ECAP + fl, CH)
                    pltpu.sync_copy(src_st.at[pl.ds(0, CH)], bsrc.at[pl.ds(fo, CH)])
                    pltpu.sync_copy(dst_st.at[pl.ds(0, CH)], bdst.at[pl.ds(fo, CH)])
                    tail_s = src_st[pl.ds(CH, 16)]
                    tail_d = dst_st[pl.ds(CH, 16)]
                    src_st[pl.ds(0, 16)] = tail_s
                    dst_st[pl.ds(0, 16)] = tail_d

                did = jnp.where(cur >= CH, CH, 0)
                out.extend([cur - did, fl + did])
            return tuple(out)

        return lax.fori_loop(0, SCH // 16, group, carry)

    cur0, fl0, cur1, fl1 = lax.fori_loop(0, E // SCH, scan_chunk, (0, 0, 0, 0))

    totals = []
    for r, (cur, fl) in enumerate(((cur0, fl0), (cur1, fl1))):
        b = w * 2 + r
        src_st, dst_st = stages[r]
        trash_src = jnp.bitwise_and(b * 13 + lane * 61, 8191)
        trash_dst = jnp.full((16,), NBW, jnp.int32)
        for t in range(4):
            base16 = cur + t * 16
            plsc.store_scatter(src_st, [base16 + lane], trash_src)
            plsc.store_scatter(dst_st, [base16 + lane], trash_dst)
        cur_p = ((cur + WIN - 1) // WIN) * WIN

        def tail_flush(k, _):
            ko = pl.multiple_of(k * 64, 64)
            to = pl.multiple_of(b * ECAP + fl + k * 64, 64)
            pltpu.sync_copy(src_st.at[pl.ds(ko, 64)], bsrc.at[pl.ds(to, 64)])
            pltpu.sync_copy(dst_st.at[pl.ds(ko, 64)], bdst.at[pl.ds(to, 64)])
            return 0

        lax.fori_loop(0, cur_p // 64, tail_flush, 0)
        totals.append(fl + cur_p)

    cnt_v[...] = jnp.full((16,), totals[0], jnp.int32)
    pltpu.sync_copy(cnt_v, woff.at[w * 2])
    cnt_v[...] = jnp.full((16,), totals[1], jnp.int32)
    pltpu.sync_copy(cnt_v, woff.at[w * 2 + 1])


@functools.partial(
    pl.kernel,
    mesh=_mesh,
    compiler_params=_sc_params,
    out_type=[
        jax.ShapeDtypeStruct((NB * ECAP + 4096,), jnp.int32),
        jax.ShapeDtypeStruct((NB * ECAP + 4096,), jnp.int32),
        jax.ShapeDtypeStruct((NB, 16), jnp.int32),
    ],
    scratch_types=[
        pltpu.VMEM((SCH,), jnp.int32),
        pltpu.VMEM((SCH,), jnp.int32),
        pltpu.VMEM((CH + 96,), jnp.int32),
        pltpu.VMEM((CH + 96,), jnp.int32),
        pltpu.VMEM((CH + 96,), jnp.int32),
        pltpu.VMEM((CH + 96,), jnp.int32),
        pltpu.VMEM((16,), jnp.int32),
    ],
)
def _bin_kernel(ei, bsrc, bdst, woff, srcs_v, dsts_v,
                src_st0, dst_st0, src_st1, dst_st1, cnt_v):
    _bin_body(ei, bsrc, bdst, woff, srcs_v, dsts_v,
              src_st0, dst_st0, src_st1, dst_st1, cnt_v)


def _agg_body(stab, bsrc, bdst, woff, So, S2o, MNo, MXo, CNTo,
              woff_v, srcc, dstc, rows, accS, accS2, accMN, accMX, acc_cnt, sem):
    w = _wid()
    ones = jnp.ones((16,), jnp.float32)
    z16 = jnp.zeros((16,), jnp.float32)

    for r in range(2):
        b = w * 2 + r
        base_e = b * ECAP
        nbase = b * NBW
        pltpu.sync_copy(woff.at[b], woff_v)
        nte = woff_v[...][0]

        def init_row(rr, _):
            for cc in range(8):
                sl = pl.ds(cc * 16, 16)
                accS[rr, sl] = z16
                accS2[rr, sl] = z16
                accMN[rr, sl] = z16 + INF
                accMX[rr, sl] = z16 - INF
            acc_cnt[rr, :] = z16
            return 0

        lax.fori_loop(0, NBW + 1, init_row, 0)

        def chunk(ci, _):
            cbase = pl.multiple_of(ci * CH, CH)
            co = pl.multiple_of(base_e + cbase, CH)
            pltpu.sync_copy(bsrc.at[pl.ds(co, CH)], srcc)
            pltpu.sync_copy(bdst.at[pl.ds(co, CH)], dstc)
            csize = jnp.minimum(CH, nte - cbase)

            def window(wi, _):
                wb = wi * WIN
                pltpu.async_copy(stab.at[srcc.at[pl.ds(wb, WIN)]], rows, sem).wait()

                def group(g, _):
                    dv = dstc[pl.ds(wb + g * 16, 16)]
                    for j in range(16):
                        d = dv[j]
                        er = g * 16 + j
                        for cc in range(8):
                            sl = pl.ds(cc * 16, 16)
                            v = rows[er, sl]
                            plsc.addupdate(accS.at[d, sl], v)
                            plsc.addupdate(accS2.at[d, sl], v * v)
                            mn = accMN[d, sl]
                            accMN[d, sl] = jnp.minimum(mn, v)
                            mx = accMX[d, sl]
                            accMX[d, sl] = jnp.maximum(mx, v)
                        plsc.addupdate(acc_cnt.at[d, :], ones)
                    return 0

                return lax.fori_loop(0, 4, group, 0)

            return lax.fori_loop(0, csize // WIN, window, 0)

        lax.fori_loop(0, (nte + CH - 1) // CH, chunk, 0)

        pltpu.sync_copy(accS.at[pl.ds(0, NBW), :], So.at[pl.ds(nbase, NBW), :])
        pltpu.sync_copy(accS2.at[pl.ds(0, NBW), :], S2o.at[pl.ds(nbase, NBW), :])
        pltpu.sync_copy(accMN.at[pl.ds(0, NBW), :], MNo.at[pl.ds(nbase, NBW), :])
        pltpu.sync_copy(accMX.at[pl.ds(0, NBW), :], MXo.at[pl.ds(nbase, NBW), :])
        pltpu.sync_copy(acc_cnt.at[pl.ds(0, NBW), :], CNTo.at[pl.ds(nbase, NBW), :])


@functools.partial(
    pl.kernel,
    mesh=_mesh,
    compiler_params=_sc_params,
    out_type=[jax.ShapeDtypeStruct((NPAD, F), jnp.float32)] * 4
    + [jax.ShapeDtypeStruct((NPAD, 16), jnp.float32)],
    scratch_types=[
        pltpu.VMEM((16,), jnp.int32),
        pltpu.VMEM((CH,), jnp.int32),
        pltpu.VMEM((CH,), jnp.int32),
        pltpu.VMEM((WIN, F), jnp.float32),
        pltpu.VMEM((NBW + 1, F), jnp.float32),
        pltpu.VMEM((NBW + 1, F), jnp.float32),
        pltpu.VMEM((NBW + 1, F), jnp.float32),
        pltpu.VMEM((NBW + 1, F), jnp.float32),
        pltpu.VMEM((NBW + 1, 16), jnp.float32),
        pltpu.SemaphoreType.DMA,
    ],
)
def _agg_kernel(stab, bsrc, bdst, woff, *rest):
    _agg_body(stab, bsrc, bdst, woff, *rest)


def _pre_tc(x_ref, w_ref, b_ref, dsup, ssup, ddem, sdem):
    t = x_ref[...] @ w_ref[...] + b_ref[...]
    dsup[...] = t[:, 0:128]
    ssup[...] = t[:, 128:256]
    ddem[...] = t[:, 256:384]
    sdem[...] = t[:, 384:512]


def _pre(x, p_sup, p_dem):
    Wcat = jnp.concatenate(
        [p_sup['pre_W'][:F], p_sup['pre_W'][F:], p_dem['pre_W'][:F], p_dem['pre_W'][F:]], axis=1)
    bcat = jnp.concatenate(
        [p_sup['pre_b'], jnp.zeros((F,), jnp.float32), p_dem['pre_b'], jnp.zeros((F,), jnp.float32)])[None, :]
    bn = 400
    f_spec = pl.BlockSpec((bn, F), lambda i: (i, 0))
    return pl.pallas_call(
        _pre_tc,
        grid=(N // bn,),
        in_specs=[
            f_spec,
            pl.BlockSpec((F, 512), lambda i: (0, 0)),
            pl.BlockSpec((1, 512), lambda i: (0, 0)),
        ],
        out_specs=[f_spec] * 4,
        out_shape=[jax.ShapeDtypeStruct((N, F), jnp.float32)] * 4,
    )(x, Wcat, bcat)


def _post_tc(x_ref, d_ref, s_ref, s2_ref, mn_ref, mx_ref, cnt_ref,
             pw_ref, pb_ref, lw_ref, lb_ref, h_ref):
    x = x_ref[...]
    d = d_ref[...]
    S = s_ref[...]
    S2 = s2_ref[...]
    MN = mn_ref[...]
    MX = mx_ref[...]
    cnt = cnt_ref[...][:, 0:1]
    cnt_c = jnp.maximum(cnt, 1.0)
    inv = 1.0 / cnt_c
    sum_h = cnt * d + S
    mean = sum_h * inv
    mean2 = (cnt * d * d + 2.0 * d * S + S2) * inv
    std = jnp.sqrt(jax.nn.relu(mean2 - mean * mean) + 1e-5)
    mnv = d + MN
    mxv = d + MX
    mnv = jnp.where(jnp.isfinite(mnv), mnv, 0.0)
    mxv = jnp.where(jnp.isfinite(mxv), mxv, 0.0)
    logdeg = jnp.log(cnt_c + 1.0)
    amp = logdeg * (1.0 / AVG_LOG)
    att = AVG_LOG / logdeg
    pw = pw_ref[...]
    out = x @ pw[0:F, :]
    for a, part in enumerate([mean, sum_h, std, mnv, mxv]):
        out += part @ pw[F + a * F:F + (a + 1) * F, :]
        out += amp * (part @ pw[F + 640 + a * F:F + 640 + (a + 1) * F, :])
        out += att * (part @ pw[F + 1280 + a * F:F + 1280 + (a + 1) * F, :])
    out = out + pb_ref[...]
    h_ref[...] = jax.nn.relu(out @ lw_ref[...] + lb_ref[...])


def _post(x, d, aggs, p):
    bn = 400
    S, S2, MN, MX, cnt = aggs
    f_spec = pl.BlockSpec((bn, F), lambda i: (i, 0))
    return pl.pallas_call(
        _post_tc,
        grid=(N // bn,),
        in_specs=[f_spec] * 6 + [
            pl.BlockSpec((bn, 16), lambda i: (i, 0)),
            pl.BlockSpec((16 * F, F), lambda i: (0, 0)),
            pl.BlockSpec((1, F), lambda i: (0, 0)),
            pl.BlockSpec((F, F), lambda i: (0, 0)),
            pl.BlockSpec((1, F), lambda i: (0, 0)),
        ],
        out_specs=f_spec,
        out_shape=jax.ShapeDtypeStruct((N, F), jnp.float32),
    )(x, d, S, S2, MN, MX, cnt,
      p['post_W'], p['post_b'][None, :], p['lin_W'], p['lin_b'][None, :])


def _combine_tc(x_ref, hi_ref, ho_ref, w_ref, b_ref, o_ref):
    w = w_ref[...]
    o = x_ref[...] @ w[0:F, :] + hi_ref[...] @ w[F:2 * F, :] + ho_ref[...] @ w[2 * F:, :]
    o_ref[...] = jax.nn.relu(o + b_ref[...])


def _combine(x, h_in, h_out, p):
    bn = 400
    f_spec = pl.BlockSpec((bn, F), lambda i: (i, 0))
    return pl.pallas_call(
        _combine_tc,
        grid=(N // bn,),
        in_specs=[f_spec, f_spec, f_spec,
                  pl.BlockSpec((3 * F, F), lambda i: (0, 0)),
                  pl.BlockSpec((1, F), lambda i: (0, 0))],
        out_specs=f_spec,
        out_shape=jax.ShapeDtypeStruct((N, F), jnp.float32),
    )(x, h_in, h_out, p['W'], p['b'][None, :])


def _final_tc(x_ref, w_ref, b_ref, o_ref):
    o_ref[...] = x_ref[...] @ w_ref[...] + b_ref[...]


def _final(x, W, b):
    Wp = jnp.zeros((F, 128), jnp.float32).at[:, 0].set(W[:, 0])
    bp = jnp.zeros((1, 128), jnp.float32).at[0, 0].set(b[0])
    bn = 400
    out = pl.pallas_call(
        _final_tc,
        grid=(N // bn,),
        in_specs=[
            pl.BlockSpec((bn, F), lambda i: (i, 0)),
            pl.BlockSpec((F, 128), lambda i: (0, 0)),
            pl.BlockSpec((1, 128), lambda i: (0, 0)),
        ],
        out_specs=pl.BlockSpec((bn, 128), lambda i: (i, 0)),
        out_shape=jax.ShapeDtypeStruct((N, 128), jnp.float32),
    )(x, Wp, bp)
    return out[:, 0]


def kernel(x, edge_index_in, edge_index_out, params):
    ein = edge_index_in.astype(jnp.int32)
    eout = edge_index_out.astype(jnp.int32)
    bins = {'in': _bin_kernel(ein), 'out': _bin_kernel(eout)}
    for layer in params['layers']:
        dsup, ssup, ddem, sdem = _pre(x, layer['supply'], layer['demand'])
        aggs_in = _agg_kernel(ssup, *bins['in'])
        aggs_out = _agg_kernel(sdem, *bins['out'])
        h_in = _post(x, dsup, aggs_in, layer['supply'])
        h_out = _post(x, ddem, aggs_out, layer['demand'])
        x = _combine(x, h_in, h_out, layer['combine'])
    return _final(x, params['out']['W'], params['out']['b'])
